# baseline (speedup 1.0000x reference)
import jax
import jax.numpy as jnp
from jax import lax
from jax.experimental import pallas as pl
from jax.experimental.pallas import tpu as pltpu
from jax.experimental.pallas import tpu_sc as plsc

N_USERS = 25000
N_ITEMS = 25000
N = N_USERS + N_ITEMS
H = 32
N_LAYERS = 3
E = 800000

GCHUNK = 64
SCHUNK = 128
BROWS = 8
EPB = BROWS * SCHUNK
CPB = EPB // GCHUNK
NSLOT = 4
DEPTH = 2
TILES = 16
CORES = 2
NBLK = 49
G = NBLK * BROWS
E_PAD = G * SCHUNK * TILES
R = E_PAD // SCHUNK

NPT = 3200
N_PAD = NPT * TILES
ZROWS = 40
ZITER = NPT // ZROWS
ZWAVE = 5
MCHUNKS = N_USERS // ZROWS

_FMT = plsc.PackFormat.INTERLEAVED


def _body(tab0, src2d, dst2d, val2d, out_users, out_items, tabs,
          acc, srcv, dstv, valv, rows_bf, rows_f,
          ma, mb0, mb1, mb2, mb3, wf, wb,
          gsem, ssem, isem, msem):
    c = lax.axis_index("c")
    s = lax.axis_index("s")

    def zinit(i, _):
        ma[i, pl.ds(0, 16)] = jnp.zeros((16,), jnp.float32)
        ma[i, pl.ds(16, 16)] = jnp.zeros((16,), jnp.float32)
        return 0
    lax.fori_loop(0, ZROWS, zinit, 0)

    def idx_block_copies(bl, slot, copy_fn):
        row = s * G + bl * BROWS
        out = []
        out.append(copy_fn(src2d.at[pl.ds(row, BROWS)], srcv.at[slot], isem))
        out.append(copy_fn(dst2d.at[pl.ds(row, BROWS)], dstv.at[slot], isem))
        out.append(copy_fn(val2d.at[pl.ds(row, BROWS)], valv.at[slot], isem))
        return out

    def offset_src(slot):
        off = jnp.full((16,), c * N_PAD, jnp.int32)
        for r in range(BROWS):
            def add16(k, _):
                srcv[slot, r, pl.ds(k * 16, 16)] = (
                    srcv[slot, r, pl.ds(k * 16, 16)] + off)
                return 0
            lax.fori_loop(0, SCHUNK // 16, add16, 0)

    def zero_stripe():
        def wave(w, _):
            base = s * NPT + w * ZWAVE * ZROWS
            for i in range(ZWAVE):
                pltpu.async_copy(ma, acc.at[pl.ds(base + i * ZROWS, ZROWS)],
                                 msem)
            for i in range(ZWAVE):
                pltpu.make_async_copy(
                    ma, acc.at[pl.ds(base + i * ZROWS, ZROWS)], msem).wait()
            return 0
        lax.fori_loop(0, ZITER // ZWAVE, wave, 0)

    def run_layer(tab, out_tab):
        def gref(idx_slot, jj):
            r, half = jj // 2, jj % 2
            idx = srcv.at[idx_slot].at[r].at[pl.ds(half * GCHUNK, GCHUNK)]
            return tab.at[idx]

        def fire_gather(idx_slot, jj, bslot):
            pltpu.async_copy(gref(idx_slot, jj), rows_bf.at[bslot],
                             gsem.at[bslot])

        def wait_gather(idx_slot, jj, bslot):
            pltpu.make_async_copy(gref(idx_slot, jj), rows_bf.at[bslot],
                                  gsem.at[bslot]).wait()

        def fire_scatter(idx_slot, r, fslot):
            pltpu.async_copy(rows_f.at[fslot],
                             acc.at[dstv.at[idx_slot].at[r]],
                             ssem.at[fslot], add=True)

        def wait_scatter(idx_slot, r, fslot):
            pltpu.make_async_copy(rows_f.at[fslot],
                                  acc.at[dstv.at[idx_slot].at[r]],
                                  ssem.at[fslot]).wait()

        def scale(idx_slot, jj, bslot, fslot):
            r, half = jj // 2, jj % 2
            eo = half * GCHUNK

            def scq(q, _):
                vv = valv[idx_slot, r, pl.ds(half * GCHUNK + q * 16, 16)]
                for e16 in range(16):
                    v = lax.gather(
                        vv, jnp.full((16, 1), e16, jnp.int32),
                        lax.GatherDimensionNumbers(
                            offset_dims=(), collapsed_slice_dims=(0,),
                            start_index_map=(0,)),
                        (1,),
                        mode=lax.GatherScatterMode.PROMISE_IN_BOUNDS)
                    e = q * 16 + e16
                    ab = rows_bf[bslot, e, pl.ds(0, 2 * 16)]
                    a, b = plsc.unpack(ab, format=_FMT)
                    rows_f[fslot, eo + e, pl.ds(0, 16)] = a * v
                    rows_f[fslot, eo + e, pl.ds(16, 16)] = b * v
                return 0
            lax.fori_loop(0, GCHUNK // 16, scq, 0)

        zero_stripe()
        plsc.subcore_barrier()

        def sync3(src, dst, sem):
            pltpu.sync_copy(src, dst)
        idx_block_copies(0, 0, sync3)
        offset_src(0)
        for j0 in range(DEPTH):
            fire_gather(0, j0, j0)

        def block(bl, _):
            cb = bl % 2
            nb = 1 - cb
            for jj in range(CPB):
                r, half = jj // 2, jj % 2
                bslot = jj % NSLOT
                fslot = r % 2

                if jj == 0:
                    @pl.when(bl >= 1)
                    def _():
                        wait_scatter(nb, BROWS - 2, 0)
                        wait_scatter(nb, BROWS - 1, 1)
                    @pl.when(bl < NBLK - 1)
                    def _():
                        idx_block_copies(bl + 1, nb, pltpu.async_copy)
                elif half == 0 and r >= 2:
                    wait_scatter(cb, r - 2, fslot)

                if jj + DEPTH < CPB:
                    fire_gather(cb, jj + DEPTH, (jj + DEPTH) % NSLOT)
                else:
                    @pl.when(bl < NBLK - 1)
                    def _():
                        if jj == CPB - DEPTH:
                            for d in idx_block_copies(bl + 1, nb,
                                                      pltpu.make_async_copy):
                                d.wait()
                            offset_src(nb)
                        fire_gather(nb, jj + DEPTH - CPB,
                                    (jj + DEPTH) % NSLOT)

                wait_gather(cb, jj, bslot)
                scale(cb, jj, bslot, fslot)
                if half == 1:
                    fire_scatter(cb, r, fslot)
            return 0
        lax.fori_loop(0, NBLK, block, 0)
        wait_scatter(0, BROWS - 2, 0)
        wait_scatter(0, BROWS - 1, 1)
        plsc.subcore_barrier()

        def wo_load(z, sl):
            return pltpu.make_async_copy(
                acc.at[pl.ds(s * NPT + z * ZROWS, ZROWS)], wf.at[sl], msem)

        def wo_store(z, sl):
            return pltpu.make_async_copy(
                wb.at[sl],
                out_tab.at[pl.ds(c * N_PAD + s * NPT + z * ZROWS, ZROWS)],
                isem)

        pltpu.async_copy(acc.at[pl.ds(s * NPT, ZROWS)], wf.at[0], msem)

        def wo_step(z, _):
            sl = z % 2
            wo_load(z, sl).wait()

            @pl.when(z < ZITER - 1)
            def _():
                pltpu.async_copy(
                    acc.at[pl.ds(s * NPT + (z + 1) * ZROWS, ZROWS)],
                    wf.at[1 - sl], msem)

            @pl.when(z >= 2)
            def _():
                wo_store(z - 2, sl).wait()

            def packrow(i, _):
                p = plsc.pack(wf[sl, i, pl.ds(0, 16)],
                              wf[sl, i, pl.ds(16, 16)], format=_FMT)
                wb[sl, i, pl.ds(0, 2 * 16)] = p
                return 0
            lax.fori_loop(0, ZROWS, packrow, 0)

            pltpu.async_copy(
                wb.at[sl],
                out_tab.at[pl.ds(c * N_PAD + s * NPT + z * ZROWS, ZROWS)],
                isem)
            return 0
        lax.fori_loop(0, ZITER, wo_step, 0)
        wo_store(ZITER - 2, 0).wait()
        wo_store(ZITER - 1, 1).wait()
        plsc.subcore_barrier()

    run_layer(tab0, tabs.at[0])

    def later_layer(l2, _):
        run_layer(tabs.at[l2], tabs.at[l2 + 1])
        return 0
    lax.fori_loop(0, N_LAYERS - 1, later_layer, 0)

    def mean_table(node_off, out_ref):
        nk = jnp.where(s == 0, (MCHUNKS + TILES - 1) // TILES,
                       MCHUNKS // TILES)

        def mean_chunk(k, _):
            q = s + k * TILES
            base = c * N_PAD + node_off + q * ZROWS
            d0 = pltpu.async_copy(tab0.at[pl.ds(base, ZROWS)], mb0, msem)
            d1 = pltpu.async_copy(tabs.at[0].at[pl.ds(base, ZROWS)], mb1, msem)
            d2 = pltpu.async_copy(tabs.at[1].at[pl.ds(base, ZROWS)], mb2, msem)
            d3 = pltpu.async_copy(tabs.at[2].at[pl.ds(base, ZROWS)], mb3, msem)
            d0.wait(); d1.wait(); d2.wait(); d3.wait()

            def mean_row(i, _):
                a0, b0 = plsc.unpack(mb0[i, pl.ds(0, 2 * 16)], format=_FMT)
                a1, b1 = plsc.unpack(mb1[i, pl.ds(0, 2 * 16)], format=_FMT)
                a2, b2 = plsc.unpack(mb2[i, pl.ds(0, 2 * 16)], format=_FMT)
                a3, b3 = plsc.unpack(mb3[i, pl.ds(0, 2 * 16)], format=_FMT)
                ma[i, pl.ds(0, 16)] = (a0 + a1 + a2 + a3) * 0.25
                ma[i, pl.ds(16, 16)] = (b0 + b1 + b2 + b3) * 0.25
                return 0
            lax.fori_loop(0, ZROWS, mean_row, 0)
            pltpu.sync_copy(ma, out_ref.at[pl.ds(q * ZROWS, ZROWS),
                                           pl.ds(c * H, H)])
            return 0
        lax.fori_loop(0, nk, mean_chunk, 0)

    mean_table(0, out_users)
    mean_table(N_USERS, out_items)


@jax.jit
def _run(tab0, src2d, dst2d, val2d):
    mesh = plsc.VectorSubcoreMesh(core_axis_name="c", subcore_axis_name="s",
                                  num_cores=CORES, num_subcores=TILES)
    f = pl.kernel(
        _body,
        out_type=(
            jax.ShapeDtypeStruct((N_USERS, 2 * H), jnp.float32),
            jax.ShapeDtypeStruct((N_ITEMS, 2 * H), jnp.float32),
            jax.ShapeDtypeStruct((N_LAYERS, CORES * N_PAD, H), jnp.bfloat16),
        ),
        mesh=mesh,
        scratch_types=[
            pltpu.VMEM_SHARED((N_PAD, H), jnp.float32),
            pltpu.VMEM((2, BROWS, SCHUNK), jnp.int32),
            pltpu.VMEM((2, BROWS, SCHUNK), jnp.int32),
            pltpu.VMEM((2, BROWS, SCHUNK), jnp.float32),
            pltpu.VMEM((NSLOT, GCHUNK, H), jnp.bfloat16),
            pltpu.VMEM((2, SCHUNK, H), jnp.float32),
            pltpu.VMEM((ZROWS, H), jnp.float32),
            pltpu.VMEM((ZROWS, H), jnp.bfloat16),
            pltpu.VMEM((ZROWS, H), jnp.bfloat16),
            pltpu.VMEM((ZROWS, H), jnp.bfloat16),
            pltpu.VMEM((ZROWS, H), jnp.bfloat16),
            pltpu.VMEM((2, ZROWS, H), jnp.float32),
            pltpu.VMEM((2, ZROWS, H), jnp.bfloat16),
            pltpu.SemaphoreType.DMA((NSLOT,)),
            pltpu.SemaphoreType.DMA((2,)),
            pltpu.SemaphoreType.DMA,
            pltpu.SemaphoreType.DMA,
        ],
        compiler_params=pltpu.CompilerParams(use_tc_tiling_on_sc=False,
                                             needs_layout_passes=False),
        name="lightgcn_sc",
    )
    return f(tab0, src2d, dst2d, val2d)


_PERM = [x for i in range(16) for x in (i, 16 + i)]


def kernel(user_emb, item_emb, adj_indices, adj_values):
    emb0 = jnp.concatenate([user_emb, item_emb], axis=0)
    npad = N_PAD - N
    zrows = jnp.zeros((npad, H), jnp.float32)
    tab0 = jnp.concatenate([emb0[:, :H], zrows, emb0[:, H:], zrows], axis=0)
    tab0 = tab0[:, jnp.array(_PERM)].astype(jnp.bfloat16)

    pad = E_PAD - E
    src = jnp.concatenate([adj_indices[0].astype(jnp.int32),
                           jnp.zeros((pad,), jnp.int32)])
    dst = jnp.concatenate([adj_indices[1].astype(jnp.int32),
                           jnp.zeros((pad,), jnp.int32)])
    val = jnp.concatenate([adj_values.astype(jnp.float32),
                           jnp.zeros((pad,), jnp.float32)])

    users, items, _ = _run(tab0, src.reshape(R, SCHUNK),
                           dst.reshape(R, SCHUNK), val.reshape(R, SCHUNK))
    return (users, items)


# speedup vs baseline: 1.8536x; 1.8536x over previous
import jax
import jax.numpy as jnp
from jax import lax
from jax.experimental import pallas as pl
from jax.experimental.pallas import tpu as pltpu
from jax.experimental.pallas import tpu_sc as plsc

N_USERS = 25000
N_ITEMS = 25000
N = N_USERS + N_ITEMS
H = 32
N_LAYERS = 3
E = 800000

CHUNK = 128
BROWS = 8
EPB = BROWS * CHUNK
NSLOT = 4
DEPTH = 2
TILES = 16
CORES = 2
NBLK = 49
G = NBLK * BROWS
E_PAD = G * CHUNK * TILES
R = E_PAD // CHUNK

NPT = 3200
N_PAD = NPT * TILES
ZROWS = 40
ZITER = NPT // ZROWS
ZWAVE = 5
MCHUNKS = N_USERS // ZROWS


def _body(tab0, src2d, dst2d, val2d, out_users, out_items, out_layers,
          acc, srcv, dstv, valv, rows4, ma, mb, mc, md,
          gsem, ssem, isem, msem):
    c = lax.axis_index("c")
    s = lax.axis_index("s")

    def zinit(i, _):
        ma[i, pl.ds(0, 16)] = jnp.zeros((16,), jnp.float32)
        ma[i, pl.ds(16, 16)] = jnp.zeros((16,), jnp.float32)
        return 0
    lax.fori_loop(0, ZROWS, zinit, 0)

    def idx_block_copies(bl, slot, copy_fn):
        row = s * G + bl * BROWS
        out = []
        out.append(copy_fn(src2d.at[pl.ds(row, BROWS)], srcv.at[slot], isem))
        out.append(copy_fn(dst2d.at[pl.ds(row, BROWS)], dstv.at[slot], isem))
        out.append(copy_fn(val2d.at[pl.ds(row, BROWS)], valv.at[slot], isem))
        return out

    def offset_src(slot):
        off = jnp.full((16,), c * N_PAD, jnp.int32)
        for r in range(BROWS):
            def add16(k, _):
                srcv[slot, r, pl.ds(k * 16, 16)] = (
                    srcv[slot, r, pl.ds(k * 16, 16)] + off)
                return 0
            lax.fori_loop(0, CHUNK // 16, add16, 0)

    def zero_stripe():
        def wave(w, _):
            base = s * NPT + w * ZWAVE * ZROWS
            for i in range(ZWAVE):
                pltpu.async_copy(ma, acc.at[pl.ds(base + i * ZROWS, ZROWS)],
                                 msem)
            for i in range(ZWAVE):
                pltpu.make_async_copy(
                    ma, acc.at[pl.ds(base + i * ZROWS, ZROWS)], msem).wait()
            return 0
        lax.fori_loop(0, ZITER // ZWAVE, wave, 0)

    for layer in range(N_LAYERS):
        tab = tab0 if layer == 0 else out_layers.at[layer - 1]

        def fire_gather(idx_slot, j, row_slot):
            pltpu.async_copy(tab.at[srcv.at[idx_slot].at[j]],
                             rows4.at[row_slot], gsem.at[row_slot])

        def wait_gather(idx_slot, j, row_slot):
            pltpu.make_async_copy(tab.at[srcv.at[idx_slot].at[j]],
                                  rows4.at[row_slot],
                                  gsem.at[row_slot]).wait()

        def fire_scatter(idx_slot, j, row_slot):
            pltpu.async_copy(rows4.at[row_slot],
                             acc.at[dstv.at[idx_slot].at[j]],
                             ssem.at[row_slot], add=True)

        def wait_scatter(idx_slot, j, row_slot):
            pltpu.make_async_copy(rows4.at[row_slot],
                                  acc.at[dstv.at[idx_slot].at[j]],
                                  ssem.at[row_slot]).wait()

        zero_stripe()
        plsc.subcore_barrier()

        def sync3(src, dst, sem):
            pltpu.sync_copy(src, dst)
        idx_block_copies(0, 0, sync3)
        offset_src(0)
        for j0 in range(DEPTH):
            fire_gather(0, j0, j0)

        def block(bl, _):
            cb = bl % 2
            nb = 1 - cb
            for j in range(BROWS):
                sj = j % NSLOT
                fs = (j + DEPTH) % NSLOT

                if j == 0:
                    @pl.when(bl >= 1)
                    def _():
                        wait_scatter(nb, BROWS - 2, (BROWS - 2) % NSLOT)
                        wait_scatter(nb, BROWS - 1, (BROWS - 1) % NSLOT)
                    @pl.when(bl < NBLK - 1)
                    def _():
                        idx_block_copies(bl + 1, nb, pltpu.async_copy)
                elif j >= DEPTH:
                    wait_scatter(cb, j - DEPTH, (j - DEPTH) % NSLOT)

                if j + DEPTH < BROWS:
                    fire_gather(cb, j + DEPTH, fs)
                else:
                    @pl.when(bl < NBLK - 1)
                    def _():
                        if j == BROWS - DEPTH:
                            for d in idx_block_copies(bl + 1, nb,
                                                      pltpu.make_async_copy):
                                d.wait()
                            offset_src(nb)
                        fire_gather(nb, j + DEPTH - BROWS, fs)

                wait_gather(cb, j, sj)

                def scale(q, _):
                    vv = valv[cb, j, pl.ds(q * 16, 16)]
                    for e16 in range(16):
                        v = lax.gather(
                            vv, jnp.full((16, 1), e16, jnp.int32),
                            lax.GatherDimensionNumbers(
                                offset_dims=(), collapsed_slice_dims=(0,),
                                start_index_map=(0,)),
                            (1,),
                            mode=lax.GatherScatterMode.PROMISE_IN_BOUNDS)
                        e = q * 16 + e16
                        rows4[sj, e, pl.ds(0, 16)] = (
                            rows4[sj, e, pl.ds(0, 16)] * v)
                        rows4[sj, e, pl.ds(16, 16)] = (
                            rows4[sj, e, pl.ds(16, 16)] * v)
                    return 0
                lax.fori_loop(0, CHUNK // 16, scale, 0)

                fire_scatter(cb, j, sj)
            return 0
        lax.fori_loop(0, NBLK, block, 0)
        wait_scatter(0, BROWS - 2, (BROWS - 2) % NSLOT)
        wait_scatter(0, BROWS - 1, (BROWS - 1) % NSLOT)
        plsc.subcore_barrier()

        pltpu.sync_copy(acc.at[pl.ds(s * NPT, NPT)],
                        out_layers.at[layer].at[pl.ds(c * N_PAD + s * NPT, NPT)])
        plsc.subcore_barrier()

    def mean_table(node_off, out_ref):
        nk = jnp.where(s == 0, (MCHUNKS + TILES - 1) // TILES,
                       MCHUNKS // TILES)

        def mean_chunk(k, _):
            q = s + k * TILES
            base = c * N_PAD + node_off + q * ZROWS
            d0 = pltpu.async_copy(tab0.at[pl.ds(base, ZROWS)], ma, msem)
            d1 = pltpu.async_copy(out_layers.at[0].at[pl.ds(base, ZROWS)],
                                  mb, msem)
            d2 = pltpu.async_copy(out_layers.at[1].at[pl.ds(base, ZROWS)],
                                  mc, msem)
            d3 = pltpu.async_copy(out_layers.at[2].at[pl.ds(base, ZROWS)],
                                  md, msem)
            d0.wait(); d1.wait(); d2.wait(); d3.wait()

            def mean_row(i, _):
                for lo in (0, 16):
                    m = (ma[i, pl.ds(lo, 16)] + mb[i, pl.ds(lo, 16)]
                         + mc[i, pl.ds(lo, 16)] + md[i, pl.ds(lo, 16)])
                    ma[i, pl.ds(lo, 16)] = m * 0.25
                return 0
            lax.fori_loop(0, ZROWS, mean_row, 0)
            pltpu.sync_copy(ma, out_ref.at[pl.ds(q * ZROWS, ZROWS),
                                           pl.ds(c * H, H)])
            return 0
        lax.fori_loop(0, nk, mean_chunk, 0)

    mean_table(0, out_users)
    mean_table(N_USERS, out_items)


@jax.jit
def _run(tab0, src2d, dst2d, val2d):
    mesh = plsc.VectorSubcoreMesh(core_axis_name="c", subcore_axis_name="s",
                                  num_cores=CORES, num_subcores=TILES)
    f = pl.kernel(
        _body,
        out_type=(
            jax.ShapeDtypeStruct((N_USERS, 2 * H), jnp.float32),
            jax.ShapeDtypeStruct((N_ITEMS, 2 * H), jnp.float32),
            jax.ShapeDtypeStruct((N_LAYERS, CORES * N_PAD, H), jnp.float32),
        ),
        mesh=mesh,
        scratch_types=[
            pltpu.VMEM_SHARED((N_PAD, H), jnp.float32),
            pltpu.VMEM((2, BROWS, CHUNK), jnp.int32),
            pltpu.VMEM((2, BROWS, CHUNK), jnp.int32),
            pltpu.VMEM((2, BROWS, CHUNK), jnp.float32),
            pltpu.VMEM((NSLOT, CHUNK, H), jnp.float32),
            pltpu.VMEM((ZROWS, H), jnp.float32),
            pltpu.VMEM((ZROWS, H), jnp.float32),
            pltpu.VMEM((ZROWS, H), jnp.float32),
            pltpu.VMEM((ZROWS, H), jnp.float32),
            pltpu.SemaphoreType.DMA((NSLOT,)),
            pltpu.SemaphoreType.DMA((NSLOT,)),
            pltpu.SemaphoreType.DMA,
            pltpu.SemaphoreType.DMA,
        ],
        compiler_params=pltpu.CompilerParams(use_tc_tiling_on_sc=False),
        name="lightgcn_sc",
    )
    return f(tab0, src2d, dst2d, val2d)


def kernel(user_emb, item_emb, adj_indices, adj_values):
    emb0 = jnp.concatenate([user_emb, item_emb], axis=0)
    npad = N_PAD - N
    zrows = jnp.zeros((npad, H), jnp.float32)
    tab0 = jnp.concatenate([emb0[:, :H], zrows, emb0[:, H:], zrows], axis=0)

    pad = E_PAD - E
    src = jnp.concatenate([adj_indices[0].astype(jnp.int32),
                           jnp.zeros((pad,), jnp.int32)])
    dst = jnp.concatenate([adj_indices[1].astype(jnp.int32),
                           jnp.zeros((pad,), jnp.int32)])
    val = jnp.concatenate([adj_values.astype(jnp.float32),
                           jnp.zeros((pad,), jnp.float32)])

    users, items, _ = _run(tab0, src.reshape(R, CHUNK),
                           dst.reshape(R, CHUNK), val.reshape(R, CHUNK))
    return (users, items)
